# baseline (speedup 1.0000x reference)
import functools

import jax
import jax.numpy as jnp
from jax import lax
from jax.experimental import pallas as pl
from jax.experimental.pallas import tpu as pltpu
from jax.experimental.pallas import tpu_sc as plsc

NC = 1
NS = 16
NW = NC * NS
CHUNK = 104
NBUF = 3


def _sc_scatter(feat, src4, dst4, n_pad, groups, d):
    zrows = n_pad // NS

    mesh = plsc.VectorSubcoreMesh(core_axis_name="c", subcore_axis_name="s", num_cores=1)

    @functools.partial(
        pl.kernel,
        out_type=jax.ShapeDtypeStruct((NC, n_pad, d), jnp.float32),
        mesh=mesh,
        scratch_types=[
            pltpu.VMEM((2, 2, NBUF, CHUNK), jnp.int32),
            pltpu.VMEM((NBUF, CHUNK, d), jnp.float32),
            pltpu.VMEM_SHARED((n_pad, d), jnp.float32),
            pltpu.SemaphoreType.DMA((NBUF,)),
            pltpu.SemaphoreType.DMA((2,)),
        ],
    )
    def k(feat_h, src_h, dst_h, out_h, idx_v, rows, acc, gsem, isem):
        c = lax.axis_index("c")
        s = lax.axis_index("s")
        wid = c * NS + s

        @pl.loop(0, CHUNK)
        def _(i):
            for kk in range(d // 16):
                rows[0, i, pl.ds(kk * 16, 16)] = jnp.zeros((16,), jnp.float32)

        nz = zrows // CHUNK
        for z in range(nz):
            pltpu.sync_copy(rows.at[0],
                            acc.at[pl.ds(s * zrows + z * CHUNK, CHUNK)])
        rem = zrows - nz * CHUNK
        if rem:
            pltpu.sync_copy(rows.at[0, pl.ds(0, rem)],
                            acc.at[pl.ds(s * zrows + nz * CHUNK, rem)])
        plsc.subcore_barrier()

        pltpu.sync_copy(src_h.at[wid, 0], idx_v.at[0, 0])
        pltpu.sync_copy(dst_h.at[wid, 0], idx_v.at[0, 1])
        for b in range(NBUF):
            pltpu.async_copy(feat_h.at[idx_v.at[0, 0, b]], rows.at[b],
                             gsem.at[b])
        if groups > 1:
            pltpu.async_copy(src_h.at[wid, 1], idx_v.at[1, 0], isem.at[0])
            pltpu.async_copy(dst_h.at[wid, 1], idx_v.at[1, 1], isem.at[1])

        @pl.loop(0, groups)
        def _(g):
            ib = lax.rem(g, 2)
            ib2 = lax.rem(g + 1, 2)

            @pl.when(g + 1 < groups)
            def _():
                pltpu.make_async_copy(src_h.at[wid, 0], idx_v.at[ib2, 0],
                                      isem.at[0]).wait()
                pltpu.make_async_copy(dst_h.at[wid, 0], idx_v.at[ib2, 1],
                                      isem.at[1]).wait()

            for b in range(NBUF):
                pltpu.make_async_copy(feat_h.at[idx_v.at[0, 0, 0]],
                                      rows.at[b], gsem.at[b]).wait()
                pltpu.sync_copy(rows.at[b], acc.at[idx_v.at[ib, 1, b]],
                                add=True)

                @pl.when(g + 1 < groups)
                def _():
                    pltpu.async_copy(feat_h.at[idx_v.at[ib2, 0, b]],
                                     rows.at[b], gsem.at[b])

            @pl.when(g + 2 < groups)
            def _():
                pltpu.async_copy(src_h.at[wid, g + 2], idx_v.at[ib, 0],
                                 isem.at[0])
                pltpu.async_copy(dst_h.at[wid, g + 2], idx_v.at[ib, 1],
                                 isem.at[1])

        plsc.subcore_barrier()
        pltpu.sync_copy(acc.at[pl.ds(s * zrows, zrows)],
                        out_h.at[c, pl.ds(s * zrows, zrows)])

    return k(feat, src4, dst4)


def _tc_combine(eps, feat, p0, rb):
    n, d = feat.shape

    def body(eps_ref, feat_ref, p0_ref, out_ref):
        out_ref[...] = (1.0 + eps_ref[0]) * feat_ref[...] + p0_ref[...]

    return pl.pallas_call(
        body,
        out_shape=jax.ShapeDtypeStruct((n, d), jnp.float32),
        grid=(n // rb,),
        in_specs=[
            pl.BlockSpec(memory_space=pltpu.SMEM),
            pl.BlockSpec((rb, d), lambda i: (i, 0)),
            pl.BlockSpec((rb, d), lambda i: (i, 0)),
        ],
        out_specs=pl.BlockSpec((rb, d), lambda i: (i, 0)),
    )(eps, feat, p0)


def kernel(feat, edge_index, radius, exp, eps):
    del radius, exp
    n, d = feat.shape
    e = edge_index.shape[1]

    q = -(-e // (NW * CHUNK))
    q = -(-q // NBUF) * NBUF
    groups = q // NBUF
    e_pad = NW * q * CHUNK
    n_pad = -(-(n + 1) // (NS * 8)) * (NS * 8)

    pad = e_pad - e
    src = jnp.concatenate([edge_index[0], jnp.zeros((pad,), jnp.int32)])
    dst = jnp.concatenate([edge_index[1], jnp.full((pad,), n, jnp.int32)])
    src4 = src.reshape(NW, groups, NBUF, CHUNK)
    dst4 = dst.reshape(NW, groups, NBUF, CHUNK)

    partials = _sc_scatter(feat, src4, dst4, n_pad, groups, d)
    return _tc_combine(eps, feat, partials[0, :n], rb=1000)


# speedup vs baseline: 1.1928x; 1.1928x over previous
import functools

import jax
import jax.numpy as jnp
from jax import lax
from jax.experimental import pallas as pl
from jax.experimental.pallas import tpu as pltpu
from jax.experimental.pallas import tpu_sc as plsc

NC = 1
NS = 16
NW = NC * NS
CHUNK = 56
NBUF = 4


def _sc_scatter(feat, src4, dst4, n_pad, groups, d):
    zrows = n_pad // NS

    mesh = plsc.VectorSubcoreMesh(core_axis_name="c", subcore_axis_name="s", num_cores=1)

    @functools.partial(
        pl.kernel,
        out_type=jax.ShapeDtypeStruct((NC, n_pad, d), jnp.float32),
        mesh=mesh,
        scratch_types=[
            pltpu.VMEM((2, 2, NBUF, CHUNK), jnp.int32),
            pltpu.VMEM((NBUF, CHUNK, d), jnp.float32),
            pltpu.VMEM_SHARED((n_pad, d), jnp.float32),
            pltpu.SemaphoreType.DMA((NBUF,)),
            pltpu.SemaphoreType.DMA((2,)),
        ],
    )
    def k(feat_h, src_h, dst_h, out_h, idx_v, rows, acc, gsem, isem):
        c = lax.axis_index("c")
        s = lax.axis_index("s")
        wid = c * NS + s

        @pl.loop(0, CHUNK)
        def _(i):
            for kk in range(d // 16):
                rows[0, i, pl.ds(kk * 16, 16)] = jnp.zeros((16,), jnp.float32)

        nz = zrows // CHUNK
        for z in range(nz):
            pltpu.sync_copy(rows.at[0],
                            acc.at[pl.ds(s * zrows + z * CHUNK, CHUNK)])
        rem = zrows - nz * CHUNK
        if rem:
            pltpu.sync_copy(rows.at[0, pl.ds(0, rem)],
                            acc.at[pl.ds(s * zrows + nz * CHUNK, rem)])
        plsc.subcore_barrier()

        pltpu.sync_copy(src_h.at[wid, 0], idx_v.at[0, 0])
        pltpu.sync_copy(dst_h.at[wid, 0], idx_v.at[0, 1])
        for b in range(NBUF):
            pltpu.async_copy(feat_h.at[idx_v.at[0, 0, b]], rows.at[b],
                             gsem.at[b])
        if groups > 1:
            pltpu.async_copy(src_h.at[wid, 1], idx_v.at[1, 0], isem.at[0])
            pltpu.async_copy(dst_h.at[wid, 1], idx_v.at[1, 1], isem.at[1])

        @pl.loop(0, groups)
        def _(g):
            ib = lax.rem(g, 2)
            ib2 = lax.rem(g + 1, 2)

            @pl.when(g + 1 < groups)
            def _():
                pltpu.make_async_copy(src_h.at[wid, 0], idx_v.at[ib2, 0],
                                      isem.at[0]).wait()
                pltpu.make_async_copy(dst_h.at[wid, 0], idx_v.at[ib2, 1],
                                      isem.at[1]).wait()

            for b in range(NBUF):
                pltpu.make_async_copy(feat_h.at[idx_v.at[0, 0, 0]],
                                      rows.at[b], gsem.at[b]).wait()
                pltpu.sync_copy(rows.at[b], acc.at[idx_v.at[ib, 1, b]],
                                add=True)

                @pl.when(g + 1 < groups)
                def _():
                    pltpu.async_copy(feat_h.at[idx_v.at[ib2, 0, b]],
                                     rows.at[b], gsem.at[b])

            @pl.when(g + 2 < groups)
            def _():
                pltpu.async_copy(src_h.at[wid, g + 2], idx_v.at[ib, 0],
                                 isem.at[0])
                pltpu.async_copy(dst_h.at[wid, g + 2], idx_v.at[ib, 1],
                                 isem.at[1])

        plsc.subcore_barrier()
        pltpu.sync_copy(acc.at[pl.ds(s * zrows, zrows)],
                        out_h.at[c, pl.ds(s * zrows, zrows)])

    return k(feat, src4, dst4)


def _tc_combine(eps, feat, p0, rb):
    n, d = feat.shape

    def body(eps_ref, feat_ref, p0_ref, out_ref):
        out_ref[...] = (1.0 + eps_ref[0]) * feat_ref[...] + p0_ref[...]

    return pl.pallas_call(
        body,
        out_shape=jax.ShapeDtypeStruct((n, d), jnp.float32),
        grid=(n // rb,),
        in_specs=[
            pl.BlockSpec(memory_space=pltpu.SMEM),
            pl.BlockSpec((rb, d), lambda i: (i, 0)),
            pl.BlockSpec((rb, d), lambda i: (i, 0)),
        ],
        out_specs=pl.BlockSpec((rb, d), lambda i: (i, 0)),
    )(eps, feat, p0)


def kernel(feat, edge_index, radius, exp, eps):
    del radius, exp
    n, d = feat.shape
    e = edge_index.shape[1]

    q = -(-e // (NW * CHUNK))
    q = -(-q // NBUF) * NBUF
    groups = q // NBUF
    e_pad = NW * q * CHUNK
    n_pad = -(-(n + 1) // (NS * 8)) * (NS * 8)

    pad = e_pad - e
    src = jnp.concatenate([edge_index[0], jnp.zeros((pad,), jnp.int32)])
    dst = jnp.concatenate([edge_index[1], jnp.full((pad,), n, jnp.int32)])
    src4 = src.reshape(NW, groups, NBUF, CHUNK)
    dst4 = dst.reshape(NW, groups, NBUF, CHUNK)

    partials = _sc_scatter(feat, src4, dst4, n_pad, groups, d)
    return _tc_combine(eps, feat, partials[0, :n], rb=1000)


# speedup vs baseline: 1.5640x; 1.3113x over previous
import functools

import jax
import jax.numpy as jnp
from jax import lax
from jax.experimental import pallas as pl
from jax.experimental.pallas import tpu as pltpu
from jax.experimental.pallas import tpu_sc as plsc

NC = 1
NS = 16
NW = NC * NS
CHUNK = 88
NBUF = 4


def _sc_scatter(feat, src4, dst4, n_pad, groups, d):
    zrows = n_pad // NS

    mesh = plsc.VectorSubcoreMesh(core_axis_name="c", subcore_axis_name="s", num_cores=1)

    @functools.partial(
        pl.kernel,
        out_type=jax.ShapeDtypeStruct((NC, n_pad, d), jnp.float32),
        mesh=mesh,
        scratch_types=[
            pltpu.VMEM((2, 2, NBUF, CHUNK), jnp.int32),
            pltpu.VMEM((NBUF, CHUNK, d), jnp.float32),
            pltpu.VMEM_SHARED((n_pad, d), jnp.float32),
            pltpu.SemaphoreType.DMA((NBUF,)),
            pltpu.SemaphoreType.DMA((2,)),
        ],
    )
    def k(feat_h, src_h, dst_h, out_h, idx_v, rows, acc, gsem, isem):
        c = lax.axis_index("c")
        s = lax.axis_index("s")
        wid = c * NS + s

        @pl.loop(0, CHUNK)
        def _(i):
            for kk in range(d // 16):
                rows[0, i, pl.ds(kk * 16, 16)] = jnp.zeros((16,), jnp.float32)

        nz = zrows // CHUNK
        for z in range(nz):
            pltpu.sync_copy(rows.at[0],
                            acc.at[pl.ds(s * zrows + z * CHUNK, CHUNK)])
        rem = zrows - nz * CHUNK
        if rem:
            pltpu.sync_copy(rows.at[0, pl.ds(0, rem)],
                            acc.at[pl.ds(s * zrows + nz * CHUNK, rem)])
        plsc.subcore_barrier()

        pltpu.sync_copy(src_h.at[wid, 0], idx_v.at[0, 0])
        pltpu.sync_copy(dst_h.at[wid, 0], idx_v.at[0, 1])
        for b in range(NBUF):
            pltpu.async_copy(feat_h.at[idx_v.at[0, 0, b]], rows.at[b],
                             gsem.at[b])
        if groups > 1:
            pltpu.async_copy(src_h.at[wid, 1], idx_v.at[1, 0], isem.at[0])
            pltpu.async_copy(dst_h.at[wid, 1], idx_v.at[1, 1], isem.at[1])

        @pl.loop(0, groups)
        def _(g):
            ib = lax.rem(g, 2)
            ib2 = lax.rem(g + 1, 2)

            @pl.when(g + 1 < groups)
            def _():
                pltpu.make_async_copy(src_h.at[wid, 0], idx_v.at[ib2, 0],
                                      isem.at[0]).wait()
                pltpu.make_async_copy(dst_h.at[wid, 0], idx_v.at[ib2, 1],
                                      isem.at[1]).wait()

            for b in range(NBUF):
                pltpu.make_async_copy(feat_h.at[idx_v.at[0, 0, 0]],
                                      rows.at[b], gsem.at[b]).wait()
                pltpu.sync_copy(rows.at[b], acc.at[idx_v.at[ib, 1, b]],
                                add=True)

                @pl.when(g + 1 < groups)
                def _():
                    pltpu.async_copy(feat_h.at[idx_v.at[ib2, 0, b]],
                                     rows.at[b], gsem.at[b])

            @pl.when(g + 2 < groups)
            def _():
                pltpu.async_copy(src_h.at[wid, g + 2], idx_v.at[ib, 0],
                                 isem.at[0])
                pltpu.async_copy(dst_h.at[wid, g + 2], idx_v.at[ib, 1],
                                 isem.at[1])

        plsc.subcore_barrier()
        pltpu.sync_copy(acc.at[pl.ds(s * zrows, zrows)],
                        out_h.at[c, pl.ds(s * zrows, zrows)])

    return k(feat, src4, dst4)


def _tc_combine(eps, feat, p0, rb):
    n, d = feat.shape

    def body(eps_ref, feat_ref, p0_ref, out_ref):
        out_ref[...] = (1.0 + eps_ref[0]) * feat_ref[...] + p0_ref[...]

    return pl.pallas_call(
        body,
        out_shape=jax.ShapeDtypeStruct((n, d), jnp.float32),
        grid=(n // rb,),
        in_specs=[
            pl.BlockSpec(memory_space=pltpu.SMEM),
            pl.BlockSpec((rb, d), lambda i: (i, 0)),
            pl.BlockSpec((rb, d), lambda i: (i, 0)),
        ],
        out_specs=pl.BlockSpec((rb, d), lambda i: (i, 0)),
    )(eps, feat, p0)


def kernel(feat, edge_index, radius, exp, eps):
    del radius, exp
    n, d = feat.shape
    e = edge_index.shape[1]

    q = -(-e // (NW * CHUNK))
    q = -(-q // NBUF) * NBUF
    groups = q // NBUF
    e_pad = NW * q * CHUNK
    n_pad = -(-(n + 1) // (NS * 8)) * (NS * 8)

    pad = e_pad - e
    src = jnp.concatenate([edge_index[0], jnp.zeros((pad,), jnp.int32)])
    dst = jnp.concatenate([edge_index[1], jnp.full((pad,), n, jnp.int32)])
    src4 = src.reshape(NW, groups, NBUF, CHUNK)
    dst4 = dst.reshape(NW, groups, NBUF, CHUNK)

    partials = _sc_scatter(feat, src4, dst4, n_pad, groups, d)
    return _tc_combine(eps, feat, partials[0, :n], rb=1000)
